# baseline (speedup 1.0000x reference)
import functools

import jax
import jax.numpy as jnp
from jax import lax
from jax.experimental import pallas as pl
from jax.experimental.pallas import tpu as pltpu
from jax.experimental.pallas import tpu_sc as plsc

N = 10000
E = 320000
D = 128

NC = 2
NS = 16
BATCH = 64
NB = 160
NBUF = 4
H = 2
NQ = 10
QROWS = NB // NQ
RPC = NB // NBUF // NQ
E_PAD = NC * NS * NB * BATCH
ROWS_PER_TILE = 632
N_PAD = NS * ROWS_PER_TILE

_MESH = plsc.VectorSubcoreMesh(core_axis_name="c", subcore_axis_name="s")


@functools.partial(
    pl.kernel,
    mesh=_MESH,
    out_type=jax.ShapeDtypeStruct((NC, N_PAD, D), jnp.float32),
    scratch_types=[
        pltpu.VMEM((2, QROWS, BATCH), jnp.int32),
        pltpu.VMEM((2, QROWS, BATCH), jnp.int32),
        pltpu.VMEM((NBUF, BATCH, D), jnp.float32),
        pltpu.VMEM_SHARED((N_PAD, D), jnp.float32),
        pltpu.SemaphoreType.DMA((NBUF,)),
        pltpu.SemaphoreType.DMA((NBUF,)),
        pltpu.SemaphoreType.DMA,
    ],
)
def _sc_aggregate(src_hbm, dst_hbm, feat_hbm, zeros_hbm, out_hbm,
                  src_v, dst_v, rows_v, agg_sh, gsem, ssem, isem):
    c = lax.axis_index("c")
    s = lax.axis_index("s")
    row0 = s * ROWS_PER_TILE
    rows = pl.ds(row0, ROWS_PER_TILE)

    def src_row(j):
        return src_v.at[(j // QROWS) % 2, j % QROWS]

    def dst_row(j):
        return dst_v.at[(j // QROWS) % 2, j % QROWS]

    def gather(j, b):
        return pltpu.make_async_copy(
            feat_hbm.at[src_row(j)], rows_v.at[b], gsem.at[b])

    def scatter(j, b):
        return pltpu.make_async_copy(
            rows_v.at[b], agg_sh.at[dst_row(j)], ssem.at[b])

    def load_chunk(q, slot):
        return (
            pltpu.make_async_copy(
                src_hbm.at[c, s, pl.ds(q * QROWS, QROWS)], src_v.at[slot],
                isem),
            pltpu.make_async_copy(
                dst_hbm.at[c, s, pl.ds(q * QROWS, QROWS)], dst_v.at[slot],
                isem),
        )

    for cp in load_chunk(0, 0):
        cp.start()
    pltpu.sync_copy(zeros_hbm, agg_sh.at[rows])
    for cp in load_chunk(0, 0):
        cp.wait()
    plsc.subcore_barrier()

    for t in range(NBUF):
        if t >= H:
            gather(t - H, t - H).wait()
            scatter(t - H, t - H).start(add=True)
        gather(t, t).start()

    def body(o, carry):
        nxt = o // RPC + 1

        @pl.when(jnp.logical_and(o % RPC == 1, nxt < NQ))
        def _():
            for cp in load_chunk(nxt, nxt % 2):
                cp.start()

        @pl.when(o % RPC == 0)
        def _():
            q = o // RPC
            for cp in load_chunk(q, q % 2):
                cp.wait()

        for b in range(NBUF):
            t = o * NBUF + b
            b2 = (b - H) % NBUF
            gather(t - H, b2).wait()
            scatter(t - H, b2).start(add=True)
            scatter(t - NBUF, b).wait()
            gather(t, b).start()
        return carry

    lax.fori_loop(1, NB // NBUF, body, 0)
    for t in range(NB, NB + H):
        b2 = (t - H) % NBUF
        gather(t - H, b2).wait()
        scatter(t - H, b2).start(add=True)
        scatter(t - NBUF, t % NBUF).wait()
    for j in range(NB - H, NB):
        scatter(j, j % NBUF).wait()

    plsc.subcore_barrier()

    pltpu.sync_copy(agg_sh.at[rows], out_hbm.at[c, rows])


_BLK = 1000


def _linear_body(p_ref, wt_ref, b_ref, o_ref):
    acc = p_ref[0] + p_ref[1]
    y = jnp.dot(acc, wt_ref[...], preferred_element_type=jnp.float32)
    o_ref[...] = jnp.maximum(y + b_ref[...], 0.0)


_tc_linear = pl.pallas_call(
    _linear_body,
    grid=(N // _BLK,),
    in_specs=[
        pl.BlockSpec((NC, _BLK, D), lambda i: (0, i, 0)),
        pl.BlockSpec((D, D), lambda i: (0, 0)),
        pl.BlockSpec((1, D), lambda i: (0, 0)),
    ],
    out_specs=pl.BlockSpec((_BLK, D), lambda i: (i, 0)),
    out_shape=jax.ShapeDtypeStruct((N, D), jnp.float32),
)


def kernel(feature, edge_index, W, b):
    pad = E_PAD - E
    pad_dst = N + (jnp.arange(pad, dtype=jnp.int32) % (N_PAD - N))
    src = jnp.concatenate([edge_index[0], jnp.zeros((pad,), jnp.int32)])
    dst = jnp.concatenate([edge_index[1], pad_dst])
    src_r = (src.reshape(NC, NS, NB, BATCH)
             + (jnp.arange(NC, dtype=jnp.int32) * N).reshape(NC, 1, 1, 1))
    dst_r = dst.reshape(NC, NS, NB, BATCH)
    feat2 = jnp.concatenate([feature, feature])
    zeros = jnp.zeros((ROWS_PER_TILE, D), jnp.float32)
    partials = _sc_aggregate(src_r, dst_r, feat2, zeros)
    return _tc_linear(partials, W.T, b.reshape(1, D))


# speedup vs baseline: 1.8095x; 1.8095x over previous
import functools

import jax
import jax.numpy as jnp
from jax import lax
from jax.experimental import pallas as pl
from jax.experimental.pallas import tpu as pltpu
from jax.experimental.pallas import tpu_sc as plsc

N = 10000
E = 320000
D = 128

NC = 2
NS = 16
BATCH = 64
NB_F = 240
NB_S = 80
NBUF = 4
H = 2
QROWS = 16
E_F = NS * NB_F * BATCH
E_PAD = NS * (NB_F + NB_S) * BATCH
ROWS_PER_TILE = 632
N_PAD = NS * ROWS_PER_TILE

_MESH = plsc.VectorSubcoreMesh(core_axis_name="c", subcore_axis_name="s")


@functools.partial(
    pl.kernel,
    mesh=_MESH,
    out_type=jax.ShapeDtypeStruct((NC, N_PAD, D), jnp.float32),
    scratch_types=[
        pltpu.VMEM((2, QROWS, BATCH), jnp.int32),
        pltpu.VMEM((2, QROWS, BATCH), jnp.int32),
        pltpu.VMEM((NBUF, BATCH, D), jnp.float32),
        pltpu.VMEM_SHARED((N_PAD, D), jnp.float32),
        pltpu.SemaphoreType.DMA((NBUF,)),
        pltpu.SemaphoreType.DMA((NBUF,)),
        pltpu.SemaphoreType.DMA,
    ],
)
def _sc_aggregate(srcf_hbm, dstf_hbm, srcs_hbm, dsts_hbm, feat_hbm,
                  zeros_hbm, out_hbm,
                  src_v, dst_v, rows_v, agg_sh, gsem, ssem, isem):
    c = lax.axis_index("c")
    s = lax.axis_index("s")
    rows = pl.ds(s * ROWS_PER_TILE, ROWS_PER_TILE)

    def pipe(src_hbm, dst_hbm, nb):
        nq = nb // QROWS
        rpc = nb // NBUF // nq

        def src_row(j):
            return src_v.at[(j // QROWS) % 2, j % QROWS]

        def dst_row(j):
            return dst_v.at[(j // QROWS) % 2, j % QROWS]

        def gather(j, b):
            return pltpu.make_async_copy(
                feat_hbm.at[src_row(j)], rows_v.at[b], gsem.at[b])

        def scatter(j, b):
            return pltpu.make_async_copy(
                rows_v.at[b], agg_sh.at[dst_row(j)], ssem.at[b])

        def load_chunk(q, slot):
            return (
                pltpu.make_async_copy(
                    src_hbm.at[s, pl.ds(q * QROWS, QROWS)], src_v.at[slot],
                    isem),
                pltpu.make_async_copy(
                    dst_hbm.at[s, pl.ds(q * QROWS, QROWS)], dst_v.at[slot],
                    isem),
            )

        for cp in load_chunk(0, 0):
            cp.start()
        for cp in load_chunk(0, 0):
            cp.wait()

        for t in range(NBUF):
            if t >= H:
                gather(t - H, t - H).wait()
                scatter(t - H, t - H).start(add=True)
            gather(t, t).start()

        def body(o, carry):
            nxt = o // rpc + 1

            @pl.when(jnp.logical_and(o % rpc == 1, nxt < nq))
            def _():
                for cp in load_chunk(nxt, nxt % 2):
                    cp.start()

            @pl.when(o % rpc == 0)
            def _():
                q = o // rpc
                for cp in load_chunk(q, q % 2):
                    cp.wait()

            for b in range(NBUF):
                t = o * NBUF + b
                b2 = (b - H) % NBUF
                gather(t - H, b2).wait()
                scatter(t - H, b2).start(add=True)
                scatter(t - NBUF, b).wait()
                gather(t, b).start()
            return carry

        lax.fori_loop(1, nb // NBUF, body, 0)
        for t in range(nb, nb + H):
            b2 = (t - H) % NBUF
            gather(t - H, b2).wait()
            scatter(t - H, b2).start(add=True)
            scatter(t - NBUF, t % NBUF).wait()
        for j in range(nb - H, nb):
            scatter(j, j % NBUF).wait()

    pltpu.sync_copy(zeros_hbm, agg_sh.at[rows])
    plsc.subcore_barrier()

    @pl.when(c == 0)
    def _():
        pipe(srcf_hbm, dstf_hbm, NB_F)

    @pl.when(c == 1)
    def _():
        pipe(srcs_hbm, dsts_hbm, NB_S)

    plsc.subcore_barrier()

    pltpu.sync_copy(agg_sh.at[rows], out_hbm.at[c, rows])


_BLK = 1000


def _linear_body(p_ref, wt_ref, b_ref, o_ref):
    acc = p_ref[0] + p_ref[1]
    y = jnp.dot(acc, wt_ref[...], preferred_element_type=jnp.float32)
    o_ref[...] = jnp.maximum(y + b_ref[...], 0.0)


_tc_linear = pl.pallas_call(
    _linear_body,
    grid=(N // _BLK,),
    in_specs=[
        pl.BlockSpec((NC, _BLK, D), lambda i: (0, i, 0)),
        pl.BlockSpec((D, D), lambda i: (0, 0)),
        pl.BlockSpec((1, D), lambda i: (0, 0)),
    ],
    out_specs=pl.BlockSpec((_BLK, D), lambda i: (i, 0)),
    out_shape=jax.ShapeDtypeStruct((N, D), jnp.float32),
)


def kernel(feature, edge_index, W, b):
    pad = E_PAD - E
    pad_src = jnp.arange(pad, dtype=jnp.int32) % N
    pad_dst = N + (jnp.arange(pad, dtype=jnp.int32) % (N_PAD - N))
    src = jnp.concatenate([edge_index[0], pad_src])
    dst = jnp.concatenate([edge_index[1], pad_dst])
    srcf = src[:E_F].reshape(NS, NB_F, BATCH)
    dstf = dst[:E_F].reshape(NS, NB_F, BATCH)
    srcs = src[E_F:].reshape(NS, NB_S, BATCH) + N
    dsts = dst[E_F:].reshape(NS, NB_S, BATCH)
    feat2 = jnp.concatenate([feature, feature])
    zeros = jnp.zeros((ROWS_PER_TILE, D), jnp.float32)
    partials = _sc_aggregate(srcf, dstf, srcs, dsts, feat2, zeros)
    return _tc_linear(partials, W.T, b.reshape(1, D))


# speedup vs baseline: 2.0951x; 1.1578x over previous
import functools

import jax
import jax.numpy as jnp
from jax import lax
from jax.experimental import pallas as pl
from jax.experimental.pallas import tpu as pltpu
from jax.experimental.pallas import tpu_sc as plsc

N = 10000
E = 320000
D = 128

NC = 2
NS = 16
BATCH = 64
NB_F = 192
NB_S = 128
NBUF = 4
H = 2
QROWS = 16
E_F = NS * NB_F * BATCH
E_PAD = NS * (NB_F + NB_S) * BATCH
ROWS_PER_TILE = 632
N_PAD = NS * ROWS_PER_TILE

_MESH = plsc.VectorSubcoreMesh(core_axis_name="c", subcore_axis_name="s")


@functools.partial(
    pl.kernel,
    mesh=_MESH,
    out_type=jax.ShapeDtypeStruct((NC, N_PAD, D), jnp.float32),
    scratch_types=[
        pltpu.VMEM((2, QROWS, BATCH), jnp.int32),
        pltpu.VMEM((2, QROWS, BATCH), jnp.int32),
        pltpu.VMEM((NBUF, BATCH, D), jnp.float32),
        pltpu.VMEM_SHARED((N_PAD, D), jnp.float32),
        pltpu.SemaphoreType.DMA((NBUF,)),
        pltpu.SemaphoreType.DMA((NBUF,)),
        pltpu.SemaphoreType.DMA,
    ],
)
def _sc_aggregate(srcf_hbm, dstf_hbm, srcs_hbm, dsts_hbm, feat_hbm,
                  zeros_hbm, out_hbm,
                  src_v, dst_v, rows_v, agg_sh, gsem, ssem, isem):
    c = lax.axis_index("c")
    s = lax.axis_index("s")
    rows = pl.ds(s * ROWS_PER_TILE, ROWS_PER_TILE)

    def pipe(src_hbm, dst_hbm, nb):
        nq = nb // QROWS
        rpc = nb // NBUF // nq

        def src_row(j):
            return src_v.at[(j // QROWS) % 2, j % QROWS]

        def dst_row(j):
            return dst_v.at[(j // QROWS) % 2, j % QROWS]

        def gather(j, b):
            return pltpu.make_async_copy(
                feat_hbm.at[src_row(j)], rows_v.at[b], gsem.at[b])

        def scatter(j, b):
            return pltpu.make_async_copy(
                rows_v.at[b], agg_sh.at[dst_row(j)], ssem.at[b])

        def load_chunk(q, slot):
            return (
                pltpu.make_async_copy(
                    src_hbm.at[s, pl.ds(q * QROWS, QROWS)], src_v.at[slot],
                    isem),
                pltpu.make_async_copy(
                    dst_hbm.at[s, pl.ds(q * QROWS, QROWS)], dst_v.at[slot],
                    isem),
            )

        for cp in load_chunk(0, 0):
            cp.start()
        for cp in load_chunk(0, 0):
            cp.wait()

        for t in range(NBUF):
            if t >= H:
                gather(t - H, t - H).wait()
                scatter(t - H, t - H).start(add=True)
            gather(t, t).start()

        def body(o, carry):
            nxt = o // rpc + 1

            @pl.when(jnp.logical_and(o % rpc == 1, nxt < nq))
            def _():
                for cp in load_chunk(nxt, nxt % 2):
                    cp.start()

            @pl.when(o % rpc == 0)
            def _():
                q = o // rpc
                for cp in load_chunk(q, q % 2):
                    cp.wait()

            for b in range(NBUF):
                t = o * NBUF + b
                b2 = (b - H) % NBUF
                gather(t - H, b2).wait()
                scatter(t - H, b2).start(add=True)
                scatter(t - NBUF, b).wait()
                gather(t, b).start()
            return carry

        lax.fori_loop(1, nb // NBUF, body, 0)
        for t in range(nb, nb + H):
            b2 = (t - H) % NBUF
            gather(t - H, b2).wait()
            scatter(t - H, b2).start(add=True)
            scatter(t - NBUF, t % NBUF).wait()
        for j in range(nb - H, nb):
            scatter(j, j % NBUF).wait()

    pltpu.sync_copy(zeros_hbm, agg_sh.at[rows])
    plsc.subcore_barrier()

    @pl.when(c == 0)
    def _():
        pipe(srcf_hbm, dstf_hbm, NB_F)

    @pl.when(c == 1)
    def _():
        pipe(srcs_hbm, dsts_hbm, NB_S)

    plsc.subcore_barrier()

    pltpu.sync_copy(agg_sh.at[rows], out_hbm.at[c, rows])


_BLK = 1000


def _linear_body(p_ref, wt_ref, b_ref, o_ref):
    acc = p_ref[0] + p_ref[1]
    y = jnp.dot(acc, wt_ref[...], preferred_element_type=jnp.float32)
    o_ref[...] = jnp.maximum(y + b_ref[...], 0.0)


_tc_linear = pl.pallas_call(
    _linear_body,
    grid=(N // _BLK,),
    in_specs=[
        pl.BlockSpec((NC, _BLK, D), lambda i: (0, i, 0)),
        pl.BlockSpec((D, D), lambda i: (0, 0)),
        pl.BlockSpec((1, D), lambda i: (0, 0)),
    ],
    out_specs=pl.BlockSpec((_BLK, D), lambda i: (i, 0)),
    out_shape=jax.ShapeDtypeStruct((N, D), jnp.float32),
)


def kernel(feature, edge_index, W, b):
    pad = E_PAD - E
    pad_src = jnp.arange(pad, dtype=jnp.int32) % N
    pad_dst = N + (jnp.arange(pad, dtype=jnp.int32) % (N_PAD - N))
    src = jnp.concatenate([edge_index[0], pad_src])
    dst = jnp.concatenate([edge_index[1], pad_dst])
    srcf = src[:E_F].reshape(NS, NB_F, BATCH)
    dstf = dst[:E_F].reshape(NS, NB_F, BATCH)
    srcs = src[E_F:].reshape(NS, NB_S, BATCH) + N
    dsts = dst[E_F:].reshape(NS, NB_S, BATCH)
    feat2 = jnp.concatenate([feature, feature])
    zeros = jnp.zeros((ROWS_PER_TILE, D), jnp.float32)
    partials = _sc_aggregate(srcf, dstf, srcs, dsts, feat2, zeros)
    return _tc_linear(partials, W.T, b.reshape(1, D))


# speedup vs baseline: 2.1710x; 1.0362x over previous
import functools

import jax
import jax.numpy as jnp
from jax import lax
from jax.experimental import pallas as pl
from jax.experimental.pallas import tpu as pltpu
from jax.experimental.pallas import tpu_sc as plsc

N = 10000
E = 320000
D = 128

NC = 2
NS = 16
BATCH = 64
NB_F = 176
NB_S = 144
NBUF = 4
H = 2
QROWS = 16
E_F = NS * NB_F * BATCH
E_PAD = NS * (NB_F + NB_S) * BATCH
ROWS_PER_TILE = 632
N_PAD = NS * ROWS_PER_TILE

_MESH = plsc.VectorSubcoreMesh(core_axis_name="c", subcore_axis_name="s")


@functools.partial(
    pl.kernel,
    mesh=_MESH,
    out_type=jax.ShapeDtypeStruct((NC, N_PAD, D), jnp.float32),
    scratch_types=[
        pltpu.VMEM((2, QROWS, BATCH), jnp.int32),
        pltpu.VMEM((2, QROWS, BATCH), jnp.int32),
        pltpu.VMEM((NBUF, BATCH, D), jnp.float32),
        pltpu.VMEM_SHARED((N_PAD, D), jnp.float32),
        pltpu.SemaphoreType.DMA((NBUF,)),
        pltpu.SemaphoreType.DMA((NBUF,)),
        pltpu.SemaphoreType.DMA,
    ],
)
def _sc_aggregate(srcf_hbm, dstf_hbm, srcs_hbm, dsts_hbm, feat_hbm,
                  zeros_hbm, out_hbm,
                  src_v, dst_v, rows_v, agg_sh, gsem, ssem, isem):
    c = lax.axis_index("c")
    s = lax.axis_index("s")
    rows = pl.ds(s * ROWS_PER_TILE, ROWS_PER_TILE)

    def pipe(src_hbm, dst_hbm, nb):
        nq = nb // QROWS
        rpc = nb // NBUF // nq

        def src_row(j):
            return src_v.at[(j // QROWS) % 2, j % QROWS]

        def dst_row(j):
            return dst_v.at[(j // QROWS) % 2, j % QROWS]

        def gather(j, b):
            return pltpu.make_async_copy(
                feat_hbm.at[src_row(j)], rows_v.at[b], gsem.at[b])

        def scatter(j, b):
            return pltpu.make_async_copy(
                rows_v.at[b], agg_sh.at[dst_row(j)], ssem.at[b])

        def load_chunk(q, slot):
            return (
                pltpu.make_async_copy(
                    src_hbm.at[s, pl.ds(q * QROWS, QROWS)], src_v.at[slot],
                    isem),
                pltpu.make_async_copy(
                    dst_hbm.at[s, pl.ds(q * QROWS, QROWS)], dst_v.at[slot],
                    isem),
            )

        for cp in load_chunk(0, 0):
            cp.start()
        for cp in load_chunk(0, 0):
            cp.wait()

        for t in range(NBUF):
            if t >= H:
                gather(t - H, t - H).wait()
                scatter(t - H, t - H).start(add=True)
            gather(t, t).start()

        def body(o, carry):
            nxt = o // rpc + 1

            @pl.when(jnp.logical_and(o % rpc == 1, nxt < nq))
            def _():
                for cp in load_chunk(nxt, nxt % 2):
                    cp.start()

            @pl.when(o % rpc == 0)
            def _():
                q = o // rpc
                for cp in load_chunk(q, q % 2):
                    cp.wait()

            for b in range(NBUF):
                t = o * NBUF + b
                b2 = (b - H) % NBUF
                gather(t - H, b2).wait()
                scatter(t - H, b2).start(add=True)
                scatter(t - NBUF, b).wait()
                gather(t, b).start()
            return carry

        lax.fori_loop(1, nb // NBUF, body, 0)
        for t in range(nb, nb + H):
            b2 = (t - H) % NBUF
            gather(t - H, b2).wait()
            scatter(t - H, b2).start(add=True)
            scatter(t - NBUF, t % NBUF).wait()
        for j in range(nb - H, nb):
            scatter(j, j % NBUF).wait()

    pltpu.sync_copy(zeros_hbm, agg_sh.at[rows])
    plsc.subcore_barrier()

    @pl.when(c == 0)
    def _():
        pipe(srcf_hbm, dstf_hbm, NB_F)

    @pl.when(c == 1)
    def _():
        pipe(srcs_hbm, dsts_hbm, NB_S)

    plsc.subcore_barrier()

    pltpu.sync_copy(agg_sh.at[rows], out_hbm.at[c, rows])


_BLK = 1000


def _linear_body(p_ref, wt_ref, b_ref, o_ref):
    acc = p_ref[0] + p_ref[1]
    y = jnp.dot(acc, wt_ref[...], preferred_element_type=jnp.float32)
    o_ref[...] = jnp.maximum(y + b_ref[...], 0.0)


_tc_linear = pl.pallas_call(
    _linear_body,
    grid=(N // _BLK,),
    in_specs=[
        pl.BlockSpec((NC, _BLK, D), lambda i: (0, i, 0)),
        pl.BlockSpec((D, D), lambda i: (0, 0)),
        pl.BlockSpec((1, D), lambda i: (0, 0)),
    ],
    out_specs=pl.BlockSpec((_BLK, D), lambda i: (i, 0)),
    out_shape=jax.ShapeDtypeStruct((N, D), jnp.float32),
)


def kernel(feature, edge_index, W, b):
    pad = E_PAD - E
    pad_src = jnp.arange(pad, dtype=jnp.int32) % N
    pad_dst = N + (jnp.arange(pad, dtype=jnp.int32) % (N_PAD - N))
    src = jnp.concatenate([edge_index[0], pad_src])
    dst = jnp.concatenate([edge_index[1], pad_dst])
    srcf = src[:E_F].reshape(NS, NB_F, BATCH)
    dstf = dst[:E_F].reshape(NS, NB_F, BATCH)
    srcs = src[E_F:].reshape(NS, NB_S, BATCH) + N
    dsts = dst[E_F:].reshape(NS, NB_S, BATCH)
    feat2 = jnp.concatenate([feature, feature])
    zeros = jnp.zeros((ROWS_PER_TILE, D), jnp.float32)
    partials = _sc_aggregate(srcf, dstf, srcs, dsts, feat2, zeros)
    return _tc_linear(partials, W.T, b.reshape(1, D))


# speedup vs baseline: 2.3391x; 1.0774x over previous
import functools

import jax
import jax.numpy as jnp
from jax import lax
from jax.experimental import pallas as pl
from jax.experimental.pallas import tpu as pltpu
from jax.experimental.pallas import tpu_sc as plsc

N = 10000
E = 320000
D = 128

NC = 2
NS = 16
BATCH = 64
NB_F = 160
NB_S = 160
NBUF = 4
H = 2
QROWS = 16
E_F = NS * NB_F * BATCH
E_PAD = NS * (NB_F + NB_S) * BATCH
ROWS_PER_TILE = 632
N_PAD = NS * ROWS_PER_TILE

_MESH = plsc.VectorSubcoreMesh(core_axis_name="c", subcore_axis_name="s")


@functools.partial(
    pl.kernel,
    mesh=_MESH,
    out_type=jax.ShapeDtypeStruct((NC, N_PAD, D), jnp.float32),
    scratch_types=[
        pltpu.VMEM((2, QROWS, BATCH), jnp.int32),
        pltpu.VMEM((2, QROWS, BATCH), jnp.int32),
        pltpu.VMEM((NBUF, BATCH, D), jnp.float32),
        pltpu.VMEM_SHARED((N_PAD, D), jnp.float32),
        pltpu.SemaphoreType.DMA((NBUF,)),
        pltpu.SemaphoreType.DMA((NBUF,)),
        pltpu.SemaphoreType.DMA,
    ],
)
def _sc_aggregate(srcf_hbm, dstf_hbm, srcs_hbm, dsts_hbm, feat_hbm,
                  zeros_hbm, out_hbm,
                  src_v, dst_v, rows_v, agg_sh, gsem, ssem, isem):
    c = lax.axis_index("c")
    s = lax.axis_index("s")
    rows = pl.ds(s * ROWS_PER_TILE, ROWS_PER_TILE)

    def pipe(src_hbm, dst_hbm, nb):
        nq = nb // QROWS
        rpc = nb // NBUF // nq

        def src_row(j):
            return src_v.at[(j // QROWS) % 2, j % QROWS]

        def dst_row(j):
            return dst_v.at[(j // QROWS) % 2, j % QROWS]

        def gather(j, b):
            return pltpu.make_async_copy(
                feat_hbm.at[src_row(j)], rows_v.at[b], gsem.at[b])

        def scatter(j, b):
            return pltpu.make_async_copy(
                rows_v.at[b], agg_sh.at[dst_row(j)], ssem.at[b])

        def load_chunk(q, slot):
            return (
                pltpu.make_async_copy(
                    src_hbm.at[s, pl.ds(q * QROWS, QROWS)], src_v.at[slot],
                    isem),
                pltpu.make_async_copy(
                    dst_hbm.at[s, pl.ds(q * QROWS, QROWS)], dst_v.at[slot],
                    isem),
            )

        for cp in load_chunk(0, 0):
            cp.start()
        for cp in load_chunk(0, 0):
            cp.wait()

        for t in range(NBUF):
            if t >= H:
                gather(t - H, t - H).wait()
                scatter(t - H, t - H).start(add=True)
            gather(t, t).start()

        def body(o, carry):
            nxt = o // rpc + 1

            @pl.when(jnp.logical_and(o % rpc == 1, nxt < nq))
            def _():
                for cp in load_chunk(nxt, nxt % 2):
                    cp.start()

            @pl.when(o % rpc == 0)
            def _():
                q = o // rpc
                for cp in load_chunk(q, q % 2):
                    cp.wait()

            for b in range(NBUF):
                t = o * NBUF + b
                b2 = (b - H) % NBUF
                gather(t - H, b2).wait()
                scatter(t - H, b2).start(add=True)
                scatter(t - NBUF, b).wait()
                gather(t, b).start()
            return carry

        lax.fori_loop(1, nb // NBUF, body, 0)
        for t in range(nb, nb + H):
            b2 = (t - H) % NBUF
            gather(t - H, b2).wait()
            scatter(t - H, b2).start(add=True)
            scatter(t - NBUF, t % NBUF).wait()
        for j in range(nb - H, nb):
            scatter(j, j % NBUF).wait()

    pltpu.sync_copy(zeros_hbm, agg_sh.at[rows])
    plsc.subcore_barrier()

    @pl.when(c == 0)
    def _():
        pipe(srcf_hbm, dstf_hbm, NB_F)

    @pl.when(c == 1)
    def _():
        pipe(srcs_hbm, dsts_hbm, NB_S)

    plsc.subcore_barrier()

    pltpu.sync_copy(agg_sh.at[rows], out_hbm.at[c, rows])


_BLK = 1000


def _linear_body(p_ref, wt_ref, b_ref, o_ref):
    acc = p_ref[0] + p_ref[1]
    y = jnp.dot(acc, wt_ref[...], preferred_element_type=jnp.float32)
    o_ref[...] = jnp.maximum(y + b_ref[...], 0.0)


_tc_linear = pl.pallas_call(
    _linear_body,
    grid=(N // _BLK,),
    in_specs=[
        pl.BlockSpec((NC, _BLK, D), lambda i: (0, i, 0)),
        pl.BlockSpec((D, D), lambda i: (0, 0)),
        pl.BlockSpec((1, D), lambda i: (0, 0)),
    ],
    out_specs=pl.BlockSpec((_BLK, D), lambda i: (i, 0)),
    out_shape=jax.ShapeDtypeStruct((N, D), jnp.float32),
)


def kernel(feature, edge_index, W, b):
    pad = E_PAD - E
    pad_src = jnp.arange(pad, dtype=jnp.int32) % N
    pad_dst = N + (jnp.arange(pad, dtype=jnp.int32) % (N_PAD - N))
    src = jnp.concatenate([edge_index[0], pad_src])
    dst = jnp.concatenate([edge_index[1], pad_dst])
    srcf = src[:E_F].reshape(NS, NB_F, BATCH)
    dstf = dst[:E_F].reshape(NS, NB_F, BATCH)
    srcs = src[E_F:].reshape(NS, NB_S, BATCH) + N
    dsts = dst[E_F:].reshape(NS, NB_S, BATCH)
    feat2 = jnp.concatenate([feature, feature])
    zeros = jnp.zeros((ROWS_PER_TILE, D), jnp.float32)
    partials = _sc_aggregate(srcf, dstf, srcs, dsts, feat2, zeros)
    return _tc_linear(partials, W.T, b.reshape(1, D))
